# baseline (speedup 1.0000x reference)
import functools

import jax
import jax.numpy as jnp
from jax import lax
from jax.experimental import pallas as pl
from jax.experimental.pallas import tpu as pltpu
from jax.experimental.pallas import tpu_sc as plsc

N_EDGES = 3200000
E1 = N_EDGES + 1
BATCH = 16384
PATH_LEN = 32

NC, NS = 2, 16
NW = NC * NS
NTOT = BATCH * PATH_LEN
PW = NTOT // NW
BPW = BATCH // NW
LANES = 16

_LOG1P_C = (3.511021357038846e-06, 0.9997923620654405, -0.49697743071892625,
            0.3145891739884515, -0.1887808235475876, 0.08172564528980446,
            -0.017207799230048133)


def _softplus(x):
    t = jnp.exp(-jnp.abs(x))
    p = jnp.full_like(t, _LOG1P_C[6])
    for c in (_LOG1P_C[5], _LOG1P_C[4], _LOG1P_C[3], _LOG1P_C[2],
              _LOG1P_C[1], _LOG1P_C[0]):
        p = p * t + c
    return jnp.maximum(x, 0.0) + p


def _sc_body(tp_hbm, reorder_hbm, w_hbm, a_hbm, fi_hbm, ti_hbm, dflt_hbm,
             dist_hbm, logp_hbm,
             idx_v, pidx_v, w_v, a_v, spw_v, spa_v, fi_v, ti_v, dflt_v,
             dist_v, logp_v, sem):
    wid = lax.axis_index("s") * NC + lax.axis_index("c")
    base = wid * PW
    pltpu.sync_copy(tp_hbm.at[pl.ds(base, PW)], idx_v)
    pltpu.sync_copy(fi_hbm.at[pl.ds(wid * BPW, BPW)], fi_v)
    pltpu.sync_copy(ti_hbm.at[pl.ds(wid * BPW, BPW)], ti_v)
    pltpu.sync_copy(dflt_hbm, dflt_v)
    pltpu.async_copy(reorder_hbm.at[idx_v], pidx_v, sem).wait()
    pltpu.async_copy(w_hbm.at[pidx_v], w_v, sem).wait()
    pltpu.async_copy(a_hbm.at[pidx_v], a_v, sem).wait()

    def p1(j, _):
        sl = pl.ds(j * LANES, LANES)
        m = idx_v[sl] != 0
        zero = jnp.zeros((LANES,), jnp.float32)
        spw_v[sl] = jnp.where(m, _softplus(w_v[sl]), zero)
        spa_v[sl] = jnp.where(m, _softplus(-a_v[sl]), zero)
        return 0

    lax.fori_loop(0, PW // LANES, p1, 0)

    lane = lax.iota(jnp.int32, LANES)

    def p2(g, _):
        b0 = g * LANES
        bid = (b0 + lane) * PATH_LEN

        def inner(l, carry):
            aw, aa = carry
            ids = bid + l
            aw = aw + plsc.load_gather(spw_v, [ids])
            aa = aa + plsc.load_gather(spa_v, [ids])
            return aw, aa

        zero = jnp.zeros((LANES,), jnp.float32)
        acc_w, acc_a = lax.fori_loop(0, PATH_LEN, inner, (zero, zero))
        tp0 = plsc.load_gather(idx_v, [bid])
        sl = pl.ds(b0, LANES)
        nf = (tp0 == 0) & (fi_v[sl] != ti_v[sl])
        dist_v[sl] = jnp.where(nf, dflt_v[...], acc_w)
        logp_v[sl] = -acc_a
        return 0

    lax.fori_loop(0, BPW // LANES, p2, 0)
    pltpu.sync_copy(dist_v, dist_hbm.at[pl.ds(wid * BPW, BPW)])
    pltpu.sync_copy(logp_v, logp_hbm.at[pl.ds(wid * BPW, BPW)])


_sc_kernel = functools.partial(
    pl.kernel,
    out_type=(jax.ShapeDtypeStruct((BATCH,), jnp.float32),
              jax.ShapeDtypeStruct((BATCH,), jnp.float32)),
    mesh=plsc.VectorSubcoreMesh(core_axis_name="c", subcore_axis_name="s"),
    scratch_types=[
        pltpu.VMEM((PW,), jnp.int32),
        pltpu.VMEM((PW,), jnp.int32),
        pltpu.VMEM((PW,), jnp.float32),
        pltpu.VMEM((PW,), jnp.float32),
        pltpu.VMEM((PW,), jnp.float32),
        pltpu.VMEM((PW,), jnp.float32),
        pltpu.VMEM((BPW,), jnp.int32),
        pltpu.VMEM((BPW,), jnp.int32),
        pltpu.VMEM((LANES,), jnp.float32),
        pltpu.VMEM((BPW,), jnp.float32),
        pltpu.VMEM((BPW,), jnp.float32),
        pltpu.SemaphoreType.DMA,
    ],
    compiler_params=pltpu.CompilerParams(needs_layout_passes=False),
)(_sc_body)


def kernel(edge_adjacency_logits, edge_weight_logits, default_distance,
           reorder, target_paths, from_ix, to_ix):
    tp_flat = target_paths.astype(jnp.int32).reshape(NTOT)
    dflt16 = jnp.broadcast_to(default_distance.reshape(1), (LANES,))
    return _sc_kernel(tp_flat,
                      reorder.astype(jnp.int32),
                      edge_weight_logits.reshape(E1),
                      edge_adjacency_logits.reshape(E1),
                      from_ix.astype(jnp.int32),
                      to_ix.astype(jnp.int32),
                      dflt16)


# speedup vs baseline: 1.8014x; 1.8014x over previous
import functools

import jax
import jax.numpy as jnp
from jax import lax
from jax.experimental import pallas as pl
from jax.experimental.pallas import tpu as pltpu
from jax.experimental.pallas import tpu_sc as plsc

N_EDGES = 3200000
E1 = N_EDGES + 1
BATCH = 16384
PATH_LEN = 32

NC, NS = 2, 16
NW = NC * NS
NTOT = BATCH * PATH_LEN
PW = NTOT // NW
BPW = BATCH // NW
LANES = 16

_LOG1P_C = (3.511021357038846e-06, 0.9997923620654405, -0.49697743071892625,
            0.3145891739884515, -0.1887808235475876, 0.08172564528980446,
            -0.017207799230048133)


def _softplus(x):
    t = jnp.exp(-jnp.abs(x))
    p = jnp.full_like(t, _LOG1P_C[6])
    for c in (_LOG1P_C[5], _LOG1P_C[4], _LOG1P_C[3], _LOG1P_C[2],
              _LOG1P_C[1], _LOG1P_C[0]):
        p = p * t + c
    return jnp.maximum(x, 0.0) + p


def _sc_reorder_body(tp_hbm, reorder_hbm, pidx_hbm, idx_v, pidx_v, sem):
    wid = lax.axis_index("s") * NC + lax.axis_index("c")
    base = wid * PW
    pltpu.sync_copy(tp_hbm.at[pl.ds(base, PW)], idx_v)
    pltpu.async_copy(reorder_hbm.at[idx_v], pidx_v, sem).wait()
    pltpu.sync_copy(pidx_v, pidx_hbm.at[pl.ds(base, PW)])


_sc_reorder = functools.partial(
    pl.kernel,
    out_type=jax.ShapeDtypeStruct((NTOT,), jnp.int32),
    mesh=plsc.VectorSubcoreMesh(core_axis_name="c", subcore_axis_name="s"),
    scratch_types=[
        pltpu.VMEM((PW,), jnp.int32),
        pltpu.VMEM((PW,), jnp.int32),
        pltpu.SemaphoreType.DMA,
    ],
    compiler_params=pltpu.CompilerParams(needs_layout_passes=False),
)(_sc_reorder_body)


def _sc_main_body(tp_hbm, pidx_hbm, pk_hbm, fi_hbm, ti_hbm, dflt_hbm,
                  dist_hbm, logp_hbm,
                  idx_v, pidx_v, pk_v, fi_v, ti_v, dflt_v, dist_v, logp_v,
                  sem):
    wid = lax.axis_index("s") * NC + lax.axis_index("c")
    base = wid * PW
    pltpu.sync_copy(tp_hbm.at[pl.ds(base, PW)], idx_v)
    pltpu.sync_copy(pidx_hbm.at[pl.ds(base, PW)], pidx_v)
    pltpu.sync_copy(fi_hbm.at[pl.ds(wid * BPW, BPW)], fi_v)
    pltpu.sync_copy(ti_hbm.at[pl.ds(wid * BPW, BPW)], ti_v)
    pltpu.sync_copy(dflt_hbm, dflt_v)
    pltpu.async_copy(pk_hbm.at[pidx_v], pk_v, sem).wait()

    lane = lax.iota(jnp.int32, LANES)
    zero_f = jnp.zeros((LANES,), jnp.float32)
    hi_mask = jnp.full((LANES,), -65536, jnp.int32)

    def grp(g, _):
        bid = (g * LANES + lane) * PATH_LEN

        def inner(l, carry):
            aw, aa = carry
            ids = bid + l
            m = plsc.load_gather(idx_v, [ids]) != 0
            pk = plsc.load_gather(pk_v, [ids])
            w = plsc.bitcast(pk & hi_mask, jnp.float32)
            a = plsc.bitcast(pk << 16, jnp.float32)
            aw = aw + jnp.where(m, _softplus(w), zero_f)
            aa = aa + jnp.where(m, _softplus(-a), zero_f)
            return aw, aa

        acc_w, acc_a = lax.fori_loop(0, PATH_LEN, inner, (zero_f, zero_f))
        tp0 = plsc.load_gather(idx_v, [bid])
        sl = pl.ds(g * LANES, LANES)
        nf = (tp0 == 0) & (fi_v[sl] != ti_v[sl])
        dist_v[sl] = jnp.where(nf, dflt_v[...], acc_w)
        logp_v[sl] = -acc_a
        return 0

    lax.fori_loop(0, BPW // LANES, grp, 0)
    pltpu.sync_copy(dist_v, dist_hbm.at[pl.ds(wid * BPW, BPW)])
    pltpu.sync_copy(logp_v, logp_hbm.at[pl.ds(wid * BPW, BPW)])


_sc_main = functools.partial(
    pl.kernel,
    out_type=(jax.ShapeDtypeStruct((BATCH,), jnp.float32),
              jax.ShapeDtypeStruct((BATCH,), jnp.float32)),
    mesh=plsc.VectorSubcoreMesh(core_axis_name="c", subcore_axis_name="s"),
    scratch_types=[
        pltpu.VMEM((PW,), jnp.int32),
        pltpu.VMEM((PW,), jnp.int32),
        pltpu.VMEM((PW,), jnp.int32),
        pltpu.VMEM((BPW,), jnp.int32),
        pltpu.VMEM((BPW,), jnp.int32),
        pltpu.VMEM((LANES,), jnp.float32),
        pltpu.VMEM((BPW,), jnp.float32),
        pltpu.VMEM((BPW,), jnp.float32),
        pltpu.SemaphoreType.DMA,
    ],
    compiler_params=pltpu.CompilerParams(needs_layout_passes=False),
)(_sc_main_body)


def _bf16_round(u):
    return (u + jnp.uint32(0x7FFF) + ((u >> 16) & jnp.uint32(1))) \
        & jnp.uint32(0xFFFF0000)


def kernel(edge_adjacency_logits, edge_weight_logits, default_distance,
           reorder, target_paths, from_ix, to_ix):
    tp_flat = target_paths.astype(jnp.int32).reshape(NTOT)
    pidx = _sc_reorder(tp_flat, reorder.astype(jnp.int32))
    wu = lax.bitcast_convert_type(edge_weight_logits, jnp.uint32)
    au = lax.bitcast_convert_type(edge_adjacency_logits, jnp.uint32)
    packed = _bf16_round(wu) | (_bf16_round(au) >> 16)
    packed = lax.bitcast_convert_type(packed, jnp.int32).reshape(E1)
    dflt16 = jnp.broadcast_to(default_distance.reshape(1), (LANES,))
    return _sc_main(tp_flat, pidx, packed,
                    from_ix.astype(jnp.int32),
                    to_ix.astype(jnp.int32),
                    dflt16)
